# baseline (device time: 50802 ns/iter reference)
import jax
import jax.numpy as jnp
from jax import lax
from jax.experimental import pallas as pl
from jax.experimental.pallas import tpu as pltpu

N_DEV = 16
N_LAYERS = 3
N_ROUNDS = 4
N_SEG = 2


def kernel(x, Win0, Wout0, Win1, Wout1, Win2, Wout2):
    b, d = x.shape
    bs = b // N_SEG

    def slot(layer, rnd, seg):
        return (layer * N_ROUNDS + rnd) * N_SEG + seg

    def body(x_ref, win0_ref, wout0_ref, win1_ref, wout1_ref, win2_ref,
             wout2_ref, out_ref, acc_ref, recv_ref, send_sems, recv_sems):
        my = lax.axis_index("i")
        q = my & 3
        wins = [win0_ref, win1_ref, win2_ref]
        wouts = [wout0_ref, wout1_ref, wout2_ref]

        partners = [my ^ 1, my + 3 - 2 * q, my ^ 4, my ^ 8]

        barrier_sem = pltpu.get_barrier_semaphore()
        for p in partners:
            pl.semaphore_signal(barrier_sem, inc=1, device_id=(p,),
                                device_id_type=pl.DeviceIdType.MESH)
        pl.semaphore_wait(barrier_sem, N_ROUNDS)

        def issue(layer, rnd, seg):
            s = slot(layer, rnd, seg)
            rdma = pltpu.make_async_remote_copy(
                src_ref=acc_ref.at[seg],
                dst_ref=recv_ref.at[s],
                send_sem=send_sems.at[s],
                recv_sem=recv_sems.at[s],
                device_id=(partners[rnd],),
                device_id_type=pl.DeviceIdType.MESH,
            )
            rdma.start()
            return rdma

        def wait_sum(layer, rnd, seg, rdma):
            rdma.wait()
            acc_ref[seg] = acc_ref[seg] + recv_ref[slot(layer, rnd, seg)]

        pending = [None] * N_SEG
        for layer in range(N_LAYERS):
            for seg in range(N_SEG):
                if layer == 0:
                    xin = x_ref[pl.ds(seg * bs, bs), :]
                else:
                    wait_sum(layer - 1, N_ROUNDS - 1, seg, pending[seg])
                    xin = acc_ref[seg]
                h = jnp.maximum(
                    jnp.dot(xin, wins[layer][...],
                            preferred_element_type=jnp.float32),
                    0.0,
                )
                acc_ref[seg] = jnp.dot(h, wouts[layer][...],
                                       preferred_element_type=jnp.float32)
                pending[seg] = issue(layer, 0, seg)
            for rnd in range(N_ROUNDS - 1):
                for seg in range(N_SEG):
                    wait_sum(layer, rnd, seg, pending[seg])
                    pending[seg] = issue(layer, rnd + 1, seg)

        for seg in range(N_SEG):
            wait_sum(N_LAYERS - 1, N_ROUNDS - 1, seg, pending[seg])
            out_ref[pl.ds(seg * bs, bs), :] = acc_ref[seg]

    n_slots = N_LAYERS * N_ROUNDS * N_SEG
    return pl.pallas_call(
        body,
        out_shape=jax.ShapeDtypeStruct((b, d), jnp.float32),
        in_specs=[pl.BlockSpec(memory_space=pltpu.VMEM)] * 7,
        out_specs=pl.BlockSpec(memory_space=pltpu.VMEM),
        scratch_shapes=[
            pltpu.VMEM((N_SEG, bs, d), jnp.float32),
            pltpu.VMEM((n_slots, bs, d), jnp.float32),
            pltpu.SemaphoreType.DMA((n_slots,)),
            pltpu.SemaphoreType.DMA((n_slots,)),
        ],
        compiler_params=pltpu.CompilerParams(collective_id=0),
    )(x, Win0, Wout0, Win1, Wout1, Win2, Wout2)


# device time: 42522 ns/iter; 1.1947x vs baseline; 1.1947x over previous
import jax
import jax.numpy as jnp
from jax import lax
from jax.experimental import pallas as pl
from jax.experimental.pallas import tpu as pltpu

N_DEV = 16
N_LAYERS = 3
N_ROUNDS = 2
N_PEERS = 3
N_SEG = 4


def kernel(x, Win0, Wout0, Win1, Wout1, Win2, Wout2):
    b, d = x.shape
    bs = b // N_SEG

    def slot(layer, rnd, seg, peer_slot):
        return (((layer * N_ROUNDS + rnd) * N_SEG + seg) * N_PEERS
                + peer_slot)

    def body(x_ref, win0_ref, wout0_ref, win1_ref, wout1_ref, win2_ref,
             wout2_ref, out_ref, acc_ref, recv_ref, send_sems, recv_sems):
        my = lax.axis_index("i")
        q = my & 3
        z_base = my - q
        wins = [win0_ref, win1_ref, win2_ref]
        wouts = [wout0_ref, wout1_ref, wout2_ref]

        plane_peers = [z_base + ((q + o) & 3) for o in (1, 2, 3)]
        z_peers = [(my + 4 * o) & 15 for o in (1, 2, 3)]
        rounds = [z_peers, plane_peers]

        barrier_sem = pltpu.get_barrier_semaphore()
        for p in plane_peers + z_peers:
            pl.semaphore_signal(barrier_sem, inc=1, device_id=(p,),
                                device_id_type=pl.DeviceIdType.MESH)
        pl.semaphore_wait(barrier_sem, 2 * N_PEERS)

        def issue(layer, rnd, seg):
            rdmas = []
            for oi, peer in enumerate(rounds[rnd]):
                s = slot(layer, rnd, seg, N_PEERS - 1 - oi)
                rdma = pltpu.make_async_remote_copy(
                    src_ref=acc_ref.at[seg],
                    dst_ref=recv_ref.at[s],
                    send_sem=send_sems.at[s],
                    recv_sem=recv_sems.at[s],
                    device_id=(peer,),
                    device_id_type=pl.DeviceIdType.MESH,
                )
                rdma.start()
                rdmas.append(rdma)
            return rdmas

        def wait_sum(rdmas, layer, rnd, seg):
            for rdma in rdmas:
                rdma.wait()
            base = slot(layer, rnd, seg, 0)
            acc_ref[seg] = (acc_ref[seg] + recv_ref[base]
                            + recv_ref[base + 1] + recv_ref[base + 2])

        pending_b = [None] * N_SEG
        for layer in range(N_LAYERS):
            pending_a = [None] * N_SEG
            for seg in range(N_SEG):
                if layer == 0:
                    xin = x_ref[pl.ds(seg * bs, bs), :]
                else:
                    wait_sum(pending_b[seg], layer - 1, 1, seg)
                    xin = acc_ref[seg]
                h = jnp.maximum(
                    jnp.dot(xin, wins[layer][...],
                            preferred_element_type=jnp.float32),
                    0.0,
                )
                acc_ref[seg] = jnp.dot(h, wouts[layer][...],
                                       preferred_element_type=jnp.float32)
                pending_a[seg] = issue(layer, 0, seg)
            for seg in range(N_SEG):
                wait_sum(pending_a[seg], layer, 0, seg)
                pending_b[seg] = issue(layer, 1, seg)

        for seg in range(N_SEG):
            wait_sum(pending_b[seg], N_LAYERS - 1, 1, seg)
            out_ref[pl.ds(seg * bs, bs), :] = acc_ref[seg]

    n_slots = N_LAYERS * N_ROUNDS * N_SEG * N_PEERS
    return pl.pallas_call(
        body,
        out_shape=jax.ShapeDtypeStruct((b, d), jnp.float32),
        in_specs=[pl.BlockSpec(memory_space=pltpu.VMEM)] * 7,
        out_specs=pl.BlockSpec(memory_space=pltpu.VMEM),
        scratch_shapes=[
            pltpu.VMEM((N_SEG, bs, d), jnp.float32),
            pltpu.VMEM((n_slots, bs, d), jnp.float32),
            pltpu.SemaphoreType.DMA((n_slots,)),
            pltpu.SemaphoreType.DMA((n_slots,)),
        ],
        compiler_params=pltpu.CompilerParams(collective_id=0),
    )(x, Win0, Wout0, Win1, Wout1, Win2, Wout2)
